# Initial kernel scaffold; baseline (speedup 1.0000x reference)
import jax
import jax.numpy as jnp
from jax.experimental import pallas as pl


def kernel(mem_h, mem_c, slots, h_out, c_out):
    raise NotImplementedError("write your pallas kernel here")


# speedup vs baseline: 11.8501x; 11.8501x over previous
import functools

import jax
import jax.numpy as jnp
from jax import lax
from jax.experimental import pallas as pl
from jax.experimental.pallas import tpu as pltpu
from jax.experimental.pallas import tpu_sc as plsc

L = 2
M = 100000
H = 128
B = 16384

NC = 2
NS = 16
NW = NC * NS

COPY_CHUNK = 200
CHUNKS_PER_CORE = M // COPY_CHUNK
MAX_COPY_ITERS = -(-CHUNKS_PER_CORE // NS)

GB_PER_W = (L * B) // NW
G_CHUNK = 128
N_G = GB_PER_W // G_CHUNK

SB_PER_W = B // NS
S_CHUNK = 128
N_S = SB_PER_W // S_CHUNK


def _body(memh, memc, hv, cv, idx2, src2, hin, cin, outh, outc,
          cbuf, gbuf, sbuf, idxb, srcb, sem):
  c = lax.axis_index("c")
  s = lax.axis_index("s")
  w = c * NS + s

  kmax = (c + 1) * CHUNKS_PER_CORE

  def copy_step(j, carry):
    k = c * CHUNKS_PER_CORE + s + j * NS

    @pl.when(k < kmax)
    def _():
      base = k * COPY_CHUNK
      pltpu.sync_copy(memh.at[pl.ds(base, COPY_CHUNK)], cbuf)
      pltpu.sync_copy(cbuf, outh.at[pl.ds(base, COPY_CHUNK)])
      pltpu.sync_copy(memc.at[pl.ds(base, COPY_CHUNK)], cbuf)
      pltpu.sync_copy(cbuf, outc.at[pl.ds(base, COPY_CHUNK)])

    return carry

  lax.fori_loop(0, MAX_COPY_ITERS, copy_step, 0)

  gbase = w * GB_PER_W

  def gather_step(j, carry):
    base = gbase + j * G_CHUNK
    pltpu.sync_copy(idx2.at[pl.ds(base, G_CHUNK)], idxb)
    pltpu.async_copy(memh.at[idxb], gbuf, sem).wait()
    pltpu.sync_copy(gbuf, hin.at[pl.ds(base, G_CHUNK)])
    pltpu.async_copy(memc.at[idxb], gbuf, sem).wait()
    pltpu.sync_copy(gbuf, cin.at[pl.ds(base, G_CHUNK)])
    return carry

  lax.fori_loop(0, N_G, gather_step, 0)

  plsc.subcore_barrier()

  sbase = c * B + s * SB_PER_W

  def scatter_step(j, carry):
    base = sbase + j * S_CHUNK
    pltpu.sync_copy(idx2.at[pl.ds(base, S_CHUNK)], idxb)
    pltpu.sync_copy(src2.at[pl.ds(base, S_CHUNK)], srcb)
    pltpu.async_copy(hv.at[srcb], sbuf, sem).wait()
    pltpu.async_copy(sbuf, outh.at[idxb], sem).wait()
    pltpu.async_copy(cv.at[srcb], sbuf, sem).wait()
    pltpu.async_copy(sbuf, outc.at[idxb], sem).wait()
    return carry

  lax.fori_loop(0, N_S, scatter_step, 0)


@functools.partial(
    pl.kernel,
    out_type=(
        jax.ShapeDtypeStruct((L * B, H), jnp.float32),
        jax.ShapeDtypeStruct((L * B, H), jnp.float32),
        jax.ShapeDtypeStruct((L * M, H), jnp.float32),
        jax.ShapeDtypeStruct((L * M, H), jnp.float32),
    ),
    mesh=plsc.VectorSubcoreMesh(core_axis_name="c", subcore_axis_name="s"),
    scratch_types=[
        pltpu.VMEM((COPY_CHUNK, H), jnp.float32),
        pltpu.VMEM((G_CHUNK, H), jnp.float32),
        pltpu.VMEM((S_CHUNK, H), jnp.float32),
        pltpu.VMEM((G_CHUNK,), jnp.int32),
        pltpu.VMEM((S_CHUNK,), jnp.int32),
        pltpu.SemaphoreType.DMA,
    ],
)
def _sc_kernel(*refs):
  _body(*refs)


def kernel(mem_h, mem_c, slots, h_out, c_out):
  slots = slots.astype(jnp.int32)
  iota = lax.iota(jnp.int32, B)
  winner = jnp.full((M,), -1, jnp.int32).at[slots].max(iota)
  src = winner[slots]
  idx2 = jnp.concatenate([slots, slots + M])
  src2 = jnp.concatenate([src, src + B])

  hin, cin, outh, outc = _sc_kernel(
      mem_h.reshape(L * M, H),
      mem_c.reshape(L * M, H),
      h_out.reshape(L * B, H),
      c_out.reshape(L * B, H),
      idx2,
      src2,
  )
  return (hin.reshape(L, B, H), cin.reshape(L, B, H),
          outh.reshape(L, M, H), outc.reshape(L, M, H))


# speedup vs baseline: 13.3995x; 1.1308x over previous
import functools

import jax
import jax.numpy as jnp
from jax import lax
from jax.experimental import pallas as pl
from jax.experimental.pallas import tpu as pltpu
from jax.experimental.pallas import tpu_sc as plsc

L = 2
M = 100000
H = 128
B = 16384

NC = 2
NS = 16
NW = NC * NS

COPY_CHUNK = 80
NBUF = 4
CHUNKS_PER_CORE = M // COPY_CHUNK
J_PER_SUB = -(-CHUNKS_PER_CORE // NS)
COPY_OUTER = -(-J_PER_SUB // NBUF)

GB_PER_W = (L * B) // NW
G_CHUNK = 128
N_G = GB_PER_W // G_CHUNK

SB_PER_W = B // NS
S_CHUNK = 128
N_S = SB_PER_W // S_CHUNK


def _body(memh, memc, hv, cv, idx2, src2, hin, cin, outh, outc,
          bh0, bh1, bh2, bh3, bc0, bc1, bc2, bc3,
          gbuf, sbuf, idxb, srcb,
          rsh0, rsh1, rsh2, rsh3, rsc0, rsc1, rsc2, rsc3,
          wsh0, wsh1, wsh2, wsh3, wsc0, wsc1, wsc2, wsc3, sem):
  c = lax.axis_index("c")
  s = lax.axis_index("s")
  w = c * NS + s

  streams = (
      (memh, outh, (bh0, bh1, bh2, bh3), (rsh0, rsh1, rsh2, rsh3),
       (wsh0, wsh1, wsh2, wsh3)),
      (memc, outc, (bc0, bc1, bc2, bc3), (rsc0, rsc1, rsc2, rsc3),
       (wsc0, wsc1, wsc2, wsc3)),
  )
  k0 = c * CHUNKS_PER_CORE + s
  kmax = (c + 1) * CHUNKS_PER_CORE

  def copy_iter(t, carry):
    for src, dst, bufs, rs, ws in streams:
      for b in range(NBUF):
        j = t * NBUF + b
        k = k0 + j * NS
        valid = k < kmax

        @pl.when(valid & (t > 0))
        def _(dst=dst, bufs=bufs, ws=ws, b=b):
          pltpu.make_async_copy(bufs[b], dst.at[pl.ds(0, COPY_CHUNK)],
                                ws[b]).wait()

        @pl.when(valid)
        def _(src=src, bufs=bufs, rs=rs, b=b, k=k):
          pltpu.async_copy(src.at[pl.ds(k * COPY_CHUNK, COPY_CHUNK)],
                           bufs[b], rs[b])

    for src, dst, bufs, rs, ws in streams:
      for b in range(NBUF):
        j = t * NBUF + b
        k = k0 + j * NS

        @pl.when(k < kmax)
        def _(src=src, dst=dst, bufs=bufs, rs=rs, ws=ws, b=b, k=k):
          pltpu.make_async_copy(src.at[pl.ds(0, COPY_CHUNK)], bufs[b],
                                rs[b]).wait()
          pltpu.async_copy(bufs[b], dst.at[pl.ds(k * COPY_CHUNK, COPY_CHUNK)],
                           ws[b])

    return carry

  lax.fori_loop(0, COPY_OUTER, copy_iter, 0)

  for src, dst, bufs, rs, ws in streams:
    for b in range(NBUF):
      pltpu.make_async_copy(bufs[b], dst.at[pl.ds(0, COPY_CHUNK)],
                            ws[b]).wait()

  gbase = w * GB_PER_W

  def gather_step(j, carry):
    base = gbase + j * G_CHUNK
    pltpu.sync_copy(idx2.at[pl.ds(base, G_CHUNK)], idxb)
    pltpu.async_copy(memh.at[idxb], gbuf, sem).wait()
    pltpu.sync_copy(gbuf, hin.at[pl.ds(base, G_CHUNK)])
    pltpu.async_copy(memc.at[idxb], gbuf, sem).wait()
    pltpu.sync_copy(gbuf, cin.at[pl.ds(base, G_CHUNK)])
    return carry

  lax.fori_loop(0, N_G, gather_step, 0)

  plsc.subcore_barrier()

  sbase = c * B + s * SB_PER_W

  def scatter_step(j, carry):
    base = sbase + j * S_CHUNK
    pltpu.sync_copy(idx2.at[pl.ds(base, S_CHUNK)], idxb)
    pltpu.sync_copy(src2.at[pl.ds(base, S_CHUNK)], srcb)
    pltpu.async_copy(hv.at[srcb], sbuf, sem).wait()
    pltpu.async_copy(sbuf, outh.at[idxb], sem).wait()
    pltpu.async_copy(cv.at[srcb], sbuf, sem).wait()
    pltpu.async_copy(sbuf, outc.at[idxb], sem).wait()
    return carry

  lax.fori_loop(0, N_S, scatter_step, 0)


@functools.partial(
    pl.kernel,
    out_type=(
        jax.ShapeDtypeStruct((L * B, H), jnp.float32),
        jax.ShapeDtypeStruct((L * B, H), jnp.float32),
        jax.ShapeDtypeStruct((L * M, H), jnp.float32),
        jax.ShapeDtypeStruct((L * M, H), jnp.float32),
    ),
    mesh=plsc.VectorSubcoreMesh(core_axis_name="c", subcore_axis_name="s"),
    scratch_types=(
        [pltpu.VMEM((COPY_CHUNK, H), jnp.float32) for _ in range(2 * NBUF)]
        + [
            pltpu.VMEM((G_CHUNK, H), jnp.float32),
            pltpu.VMEM((S_CHUNK, H), jnp.float32),
            pltpu.VMEM((G_CHUNK,), jnp.int32),
            pltpu.VMEM((S_CHUNK,), jnp.int32),
        ]
        + [pltpu.SemaphoreType.DMA for _ in range(4 * NBUF + 1)]
    ),
)
def _sc_kernel(*refs):
  _body(*refs)


def kernel(mem_h, mem_c, slots, h_out, c_out):
  slots = slots.astype(jnp.int32)
  iota = lax.iota(jnp.int32, B)
  winner = jnp.full((M,), -1, jnp.int32).at[slots].max(iota)
  src = winner[slots]
  idx2 = jnp.concatenate([slots, slots + M])
  src2 = jnp.concatenate([src, src + B])

  hin, cin, outh, outc = _sc_kernel(
      mem_h.reshape(L * M, H),
      mem_c.reshape(L * M, H),
      h_out.reshape(L * B, H),
      c_out.reshape(L * B, H),
      idx2,
      src2,
  )
  return (hin.reshape(L, B, H), cin.reshape(L, B, H),
          outh.reshape(L, M, H), outc.reshape(L, M, H))


# speedup vs baseline: 16.0375x; 1.1969x over previous
import functools

import jax
import jax.numpy as jnp
from jax import lax
from jax.experimental import pallas as pl
from jax.experimental.pallas import tpu as pltpu
from jax.experimental.pallas import tpu_sc as plsc

L = 2
M = 100000
H = 128
B = 16384

NC = 2
NS = 16
NW = NC * NS

COPY_BLOCK = 2000
N_BLOCKS = (L * M) // COPY_BLOCK

RB_PER_W = (L * B) // NW
CHUNK = 128
N_CH = RB_PER_W // CHUNK


def _tc_copy_body(hsrc, csrc, hdst, cdst):
  hdst[...] = hsrc[...]
  cdst[...] = csrc[...]


_tc_copy = pl.pallas_call(
    _tc_copy_body,
    grid=(N_BLOCKS,),
    in_specs=[
        pl.BlockSpec((COPY_BLOCK, H), lambda i: (i, 0)),
        pl.BlockSpec((COPY_BLOCK, H), lambda i: (i, 0)),
    ],
    out_specs=[
        pl.BlockSpec((COPY_BLOCK, H), lambda i: (i, 0)),
        pl.BlockSpec((COPY_BLOCK, H), lambda i: (i, 0)),
    ],
    out_shape=[
        jax.ShapeDtypeStruct((L * M, H), jnp.float32),
        jax.ShapeDtypeStruct((L * M, H), jnp.float32),
    ],
)


def _sc_body(memh, memc, hv, cv, idx2, src2, outh, outc, hin, cin,
             gbuf, sbuf, idxb, srcb, sem):
  c = lax.axis_index("c")
  s = lax.axis_index("s")
  w = c * NS + s
  base0 = w * RB_PER_W

  def gather_step(j, carry):
    base = base0 + j * CHUNK
    pltpu.sync_copy(idx2.at[pl.ds(base, CHUNK)], idxb)
    pltpu.async_copy(memh.at[idxb], gbuf, sem).wait()
    pltpu.sync_copy(gbuf, hin.at[pl.ds(base, CHUNK)])
    pltpu.async_copy(memc.at[idxb], gbuf, sem).wait()
    pltpu.sync_copy(gbuf, cin.at[pl.ds(base, CHUNK)])
    return carry

  lax.fori_loop(0, N_CH, gather_step, 0)

  def scatter_step(j, carry):
    base = base0 + j * CHUNK
    pltpu.sync_copy(idx2.at[pl.ds(base, CHUNK)], idxb)
    pltpu.sync_copy(src2.at[pl.ds(base, CHUNK)], srcb)
    pltpu.async_copy(hv.at[srcb], sbuf, sem).wait()
    pltpu.async_copy(sbuf, outh.at[idxb], sem).wait()
    pltpu.async_copy(cv.at[srcb], sbuf, sem).wait()
    pltpu.async_copy(sbuf, outc.at[idxb], sem).wait()
    return carry

  lax.fori_loop(0, N_CH, scatter_step, 0)


_sc_kernel = functools.partial(
    pl.kernel,
    out_type=(
        jax.ShapeDtypeStruct((L * B, H), jnp.float32),
        jax.ShapeDtypeStruct((L * B, H), jnp.float32),
    ),
    mesh=plsc.VectorSubcoreMesh(core_axis_name="c", subcore_axis_name="s"),
    scratch_types=[
        pltpu.VMEM((CHUNK, H), jnp.float32),
        pltpu.VMEM((CHUNK, H), jnp.float32),
        pltpu.VMEM((CHUNK,), jnp.int32),
        pltpu.VMEM((CHUNK,), jnp.int32),
        pltpu.SemaphoreType.DMA,
    ],
)(_sc_body)


def kernel(mem_h, mem_c, slots, h_out, c_out):
  slots = slots.astype(jnp.int32)
  iota = lax.iota(jnp.int32, B)
  winner = jnp.full((M,), -1, jnp.int32).at[slots].max(iota)
  src = winner[slots]
  idx2 = jnp.concatenate([slots, slots + M])
  src2 = jnp.concatenate([src, src + B])

  outh0, outc0 = _tc_copy(mem_h.reshape(L * M, H), mem_c.reshape(L * M, H))
  rh = jax.new_ref(outh0)
  rc = jax.new_ref(outc0)
  hin, cin = _sc_kernel(
      mem_h.reshape(L * M, H),
      mem_c.reshape(L * M, H),
      h_out.reshape(L * B, H),
      c_out.reshape(L * B, H),
      idx2,
      src2,
      rh,
      rc,
  )
  return (hin.reshape(L, B, H), cin.reshape(L, B, H),
          rh[...].reshape(L, M, H), rc[...].reshape(L, M, H))
